# speedup vs baseline: 2.0032x; 2.0032x over previous
import functools

import jax
import jax.numpy as jnp
from jax import lax
from jax.experimental import pallas as pl
from jax.experimental.pallas import tpu as pltpu
from jax.experimental.pallas import tpu_sc as plsc

_FEAT = 512
_NBUCKET = 100
_BATCH = 16384
_START_SMOOTH = 1

_NC, _NS, _L = 2, 16, 16
_NW = _NC * _NS
_RPW = _BATCH // _NW
_CH = 32
_NCHUNK = _RPW // _CH


def _tables_body(rv_ref, sv_ref, rm_ref, sm_ref, scale_ref, off_ref):
    s = jnp.sqrt(jnp.clip(sv_ref[...] / rv_ref[...], 0.1, 10.0))
    scale_ref[...] = s
    off_ref[...] = sm_ref[...] - rm_ref[...] * s


def _make_tables(rv, sv, rm, sm):
    return pl.pallas_call(
        _tables_body,
        out_shape=(
            jax.ShapeDtypeStruct((_NBUCKET, _FEAT), jnp.float32),
            jax.ShapeDtypeStruct((_NBUCKET, _FEAT), jnp.float32),
        ),
    )(rv, sv, rm, sm)


@functools.partial(
    pl.kernel,
    out_type=jax.ShapeDtypeStruct((_BATCH, _FEAT), jnp.float32),
    mesh=plsc.VectorSubcoreMesh(core_axis_name="c", subcore_axis_name="s"),
    scratch_types=[
        pltpu.VMEM((_CH,), jnp.int32),
        pltpu.VMEM((_CH, _FEAT), jnp.float32),
        pltpu.VMEM((_CH, _FEAT), jnp.float32),
        pltpu.VMEM((_CH, _FEAT), jnp.float32),
        pltpu.SemaphoreType.DMA,
    ],
)
def _sc_apply(feat_hbm, lab_hbm, scale_hbm, off_hbm, out_hbm,
              idx_v, f_v, s_v, o_v, sem):
    wid = lax.axis_index("s") * _NC + lax.axis_index("c")
    base = wid * _RPW

    def chunk_body(ci, carry):
        row0 = base + ci * _CH
        pltpu.sync_copy(lab_hbm.at[pl.ds(row0, _CH)], idx_v)
        cs = pltpu.async_copy(scale_hbm.at[idx_v], s_v, sem)
        co = pltpu.async_copy(off_hbm.at[idx_v], o_v, sem)
        pltpu.sync_copy(feat_hbm.at[pl.ds(row0, _CH)], f_v)
        cs.wait()
        co.wait()

        def row_body(r, c2):
            for j in range(_FEAT // _L):
                sl = (r, pl.ds(j * _L, _L))
                f_v[sl] = f_v[sl] * s_v[sl] + o_v[sl]
            return c2

        lax.fori_loop(0, _CH, row_body, 0)
        pltpu.sync_copy(f_v, out_hbm.at[pl.ds(row0, _CH)])
        return carry

    lax.fori_loop(0, _NCHUNK, chunk_body, 0)


def kernel(features, labels, epoch,
           running_mean_last_epoch, running_var_last_epoch,
           smoothed_mean_last_epoch, smoothed_var_last_epoch):
    lab = jnp.clip(labels.reshape(-1).astype(jnp.int32), 0, _NBUCKET - 1)

    def smoothed():
        scale, offset = _make_tables(
            running_var_last_epoch, smoothed_var_last_epoch,
            running_mean_last_epoch, smoothed_mean_last_epoch)
        return _sc_apply(features, lab, scale, offset)

    return lax.cond(epoch < _START_SMOOTH, lambda: features, smoothed)


# speedup vs baseline: 2.2551x; 1.1257x over previous
import functools

import jax
import jax.numpy as jnp
from jax import lax
from jax.experimental import pallas as pl
from jax.experimental.pallas import tpu as pltpu
from jax.experimental.pallas import tpu_sc as plsc

_FEAT = 512
_NBUCKET = 100
_BATCH = 16384
_START_SMOOTH = 1

_NC, _NS, _L = 2, 16, 16
_NW = _NC * _NS
_RPW = _BATCH // _NW
_CH = 16
_NCHUNK = _RPW // _CH
_NBUF = 2


def _tables_body(rv_ref, sv_ref, rm_ref, sm_ref, scale_ref, off_ref):
    s = jnp.sqrt(jnp.clip(sv_ref[...] / rv_ref[...], 0.1, 10.0))
    scale_ref[...] = s
    off_ref[...] = sm_ref[...] - rm_ref[...] * s


def _make_tables(rv, sv, rm, sm):
    return pl.pallas_call(
        _tables_body,
        out_shape=(
            jax.ShapeDtypeStruct((_NBUCKET, _FEAT), jnp.float32),
            jax.ShapeDtypeStruct((_NBUCKET, _FEAT), jnp.float32),
        ),
    )(rv, sv, rm, sm)


@functools.partial(
    pl.kernel,
    out_type=jax.ShapeDtypeStruct((_BATCH, _FEAT), jnp.float32),
    mesh=plsc.VectorSubcoreMesh(core_axis_name="c", subcore_axis_name="s"),
    scratch_types=[
        pltpu.VMEM((_RPW,), jnp.int32),
        pltpu.VMEM((_NBUF, _CH, _FEAT), jnp.float32),
        pltpu.VMEM((_NBUF, _CH, _FEAT), jnp.float32),
        pltpu.VMEM((_NBUF, _CH, _FEAT), jnp.float32),
        pltpu.VMEM((_NBUF, _CH, _FEAT), jnp.float32),
        pltpu.SemaphoreType.DMA,
        pltpu.SemaphoreType.DMA,
        pltpu.SemaphoreType.DMA,
        pltpu.SemaphoreType.DMA,
    ],
)
def _sc_apply(feat_hbm, lab_hbm, scale_hbm, off_hbm, out_hbm,
              idx_v, f_v, s_v, o_v, r_v, sin0, sin1, sout0, sout1):
    sin = (sin0, sin1)
    sout = (sout0, sout1)
    wid = lax.axis_index("s") * _NC + lax.axis_index("c")
    base = wid * _RPW
    pltpu.sync_copy(lab_hbm.at[pl.ds(base, _RPW)], idx_v)

    def issue_in(ci, b):
        idx = idx_v.at[pl.ds(ci * _CH, _CH)]
        pltpu.async_copy(scale_hbm.at[idx], s_v.at[b], sin[b])
        pltpu.async_copy(off_hbm.at[idx], o_v.at[b], sin[b])
        pltpu.async_copy(feat_hbm.at[pl.ds(base + ci * _CH, _CH)],
                         f_v.at[b], sin[b])

    def wait_in(b):
        pltpu.make_async_copy(scale_hbm.at[idx_v.at[pl.ds(0, _CH)]],
                              s_v.at[b], sin[b]).wait()
        pltpu.make_async_copy(off_hbm.at[idx_v.at[pl.ds(0, _CH)]],
                              o_v.at[b], sin[b]).wait()
        pltpu.make_async_copy(feat_hbm.at[pl.ds(base, _CH)],
                              f_v.at[b], sin[b]).wait()

    def wait_out(b):
        pltpu.make_async_copy(r_v.at[b], out_hbm.at[pl.ds(base, _CH)],
                              sout[b]).wait()

    for b in range(_NBUF):
        issue_in(b, b)

    def outer(ci2, carry):
        for b in range(_NBUF):
            ci = ci2 * _NBUF + b
            wait_in(b)

            @pl.when(ci2 > 0)
            def _():
                wait_out(b)

            def row_body(r, c2):
                for j in range(_FEAT // _L):
                    sl = (b, r, pl.ds(j * _L, _L))
                    r_v[sl] = f_v[sl] * s_v[sl] + o_v[sl]
                return c2

            lax.fori_loop(0, _CH, row_body, 0)
            pltpu.async_copy(r_v.at[b], out_hbm.at[pl.ds(base + ci * _CH, _CH)],
                             sout[b])

            @pl.when(ci + _NBUF < _NCHUNK)
            def _():
                issue_in(ci + _NBUF, b)
        return carry

    lax.fori_loop(0, _NCHUNK // _NBUF, outer, 0)
    for b in range(_NBUF):
        wait_out(b)


def kernel(features, labels, epoch,
           running_mean_last_epoch, running_var_last_epoch,
           smoothed_mean_last_epoch, smoothed_var_last_epoch):
    lab = jnp.clip(labels.reshape(-1).astype(jnp.int32), 0, _NBUCKET - 1)

    def smoothed():
        scale, offset = _make_tables(
            running_var_last_epoch, smoothed_var_last_epoch,
            running_mean_last_epoch, smoothed_mean_last_epoch)
        return _sc_apply(features, lab, scale, offset)

    return lax.cond(epoch < _START_SMOOTH, lambda: features, smoothed)


# speedup vs baseline: 3.1270x; 1.3866x over previous
import functools

import jax
import jax.numpy as jnp
from jax import lax
from jax.experimental import pallas as pl
from jax.experimental.pallas import tpu as pltpu
from jax.experimental.pallas import tpu_sc as plsc

_FEAT = 512
_NBUCKET = 100
_BATCH = 16384
_START_SMOOTH = 1

_NC, _NS, _L = 2, 16, 16
_NW = _NC * _NS
_RPW = _BATCH // _NW
_CH = 16
_NCHUNK = _RPW // _CH
_NBUF = 2


def _tables_body(ep_ref, rv_ref, sv_ref, rm_ref, sm_ref, scale_ref, off_ref):
    live = ep_ref[0, 0] >= _START_SMOOTH
    s = jnp.sqrt(jnp.clip(sv_ref[...] / rv_ref[...], 0.1, 10.0))
    s = jnp.where(live, s, 1.0)
    scale_ref[...] = s
    off_ref[...] = jnp.where(live, sm_ref[...] - rm_ref[...] * s, 0.0)


def _make_tables(ep, rv, sv, rm, sm):
    return pl.pallas_call(
        _tables_body,
        in_specs=[
            pl.BlockSpec(memory_space=pltpu.SMEM),
            pl.BlockSpec(memory_space=pltpu.VMEM),
            pl.BlockSpec(memory_space=pltpu.VMEM),
            pl.BlockSpec(memory_space=pltpu.VMEM),
            pl.BlockSpec(memory_space=pltpu.VMEM),
        ],
        out_shape=(
            jax.ShapeDtypeStruct((_NBUCKET, _FEAT), jnp.float32),
            jax.ShapeDtypeStruct((_NBUCKET, _FEAT), jnp.float32),
        ),
    )(ep, rv, sv, rm, sm)


@functools.partial(
    pl.kernel,
    out_type=jax.ShapeDtypeStruct((_BATCH, _FEAT), jnp.float32),
    mesh=plsc.VectorSubcoreMesh(core_axis_name="c", subcore_axis_name="s"),
    scratch_types=[
        pltpu.VMEM((_RPW,), jnp.int32),
        pltpu.VMEM((_NBUF, _CH, _FEAT), jnp.float32),
        pltpu.VMEM((_NBUF, _CH, _FEAT), jnp.float32),
        pltpu.VMEM((_NBUF, _CH, _FEAT), jnp.float32),
        pltpu.VMEM((_NBUF, _CH, _FEAT), jnp.float32),
        pltpu.SemaphoreType.DMA,
        pltpu.SemaphoreType.DMA,
        pltpu.SemaphoreType.DMA,
        pltpu.SemaphoreType.DMA,
    ],
)
def _sc_apply(feat_hbm, lab_hbm, scale_hbm, off_hbm, out_hbm,
              idx_v, f_v, s_v, o_v, r_v, sin0, sin1, sout0, sout1):
    sin = (sin0, sin1)
    sout = (sout0, sout1)
    wid = lax.axis_index("s") * _NC + lax.axis_index("c")
    base = wid * _RPW
    pltpu.sync_copy(lab_hbm.at[pl.ds(base, _RPW)], idx_v)

    def issue_in(ci, b):
        idx = idx_v.at[pl.ds(ci * _CH, _CH)]
        pltpu.async_copy(scale_hbm.at[idx], s_v.at[b], sin[b])
        pltpu.async_copy(off_hbm.at[idx], o_v.at[b], sin[b])
        pltpu.async_copy(feat_hbm.at[pl.ds(base + ci * _CH, _CH)],
                         f_v.at[b], sin[b])

    def wait_in(b):
        pltpu.make_async_copy(scale_hbm.at[idx_v.at[pl.ds(0, _CH)]],
                              s_v.at[b], sin[b]).wait()
        pltpu.make_async_copy(off_hbm.at[idx_v.at[pl.ds(0, _CH)]],
                              o_v.at[b], sin[b]).wait()
        pltpu.make_async_copy(feat_hbm.at[pl.ds(base, _CH)],
                              f_v.at[b], sin[b]).wait()

    def wait_out(b):
        pltpu.make_async_copy(r_v.at[b], out_hbm.at[pl.ds(base, _CH)],
                              sout[b]).wait()

    for b in range(_NBUF):
        issue_in(b, b)

    def outer(ci2, carry):
        for b in range(_NBUF):
            ci = ci2 * _NBUF + b
            wait_in(b)

            @pl.when(ci2 > 0)
            def _():
                wait_out(b)

            def row_body(r, c2):
                for j in range(_FEAT // _L):
                    sl = (b, r, pl.ds(j * _L, _L))
                    r_v[sl] = f_v[sl] * s_v[sl] + o_v[sl]
                return c2

            lax.fori_loop(0, _CH, row_body, 0)
            pltpu.async_copy(r_v.at[b], out_hbm.at[pl.ds(base + ci * _CH, _CH)],
                             sout[b])

            @pl.when(ci + _NBUF < _NCHUNK)
            def _():
                issue_in(ci + _NBUF, b)
        return carry

    lax.fori_loop(0, _NCHUNK // _NBUF, outer, 0)
    for b in range(_NBUF):
        wait_out(b)


def kernel(features, labels, epoch,
           running_mean_last_epoch, running_var_last_epoch,
           smoothed_mean_last_epoch, smoothed_var_last_epoch):
    lab = jnp.clip(labels.reshape(-1).astype(jnp.int32), 0, _NBUCKET - 1)
    ep = jnp.asarray(epoch, jnp.int32).reshape(1, 1)
    scale, offset = _make_tables(
        ep, running_var_last_epoch, smoothed_var_last_epoch,
        running_mean_last_epoch, smoothed_mean_last_epoch)
    return _sc_apply(features, lab, scale, offset)


# speedup vs baseline: 3.8080x; 1.2178x over previous
import functools

import jax
import jax.numpy as jnp
from jax import lax
from jax.experimental import pallas as pl
from jax.experimental.pallas import tpu as pltpu
from jax.experimental.pallas import tpu_sc as plsc

_FEAT = 512
_NBUCKET = 100
_BATCH = 16384
_START_SMOOTH = 1

_NC, _NS, _L = 2, 16, 16
_NW = _NC * _NS
_RPW = _BATCH // _NW
_CH = 16
_NCHUNK = _RPW // _CH
_NBUF = 4


def _tables_body(ep_ref, rv_ref, sv_ref, rm_ref, sm_ref, so_ref):
    live = ep_ref[0, 0] >= _START_SMOOTH
    s = jnp.sqrt(jnp.clip(sv_ref[...] / rv_ref[...], 0.1, 10.0))
    s = jnp.where(live, s, 1.0)
    o = jnp.where(live, sm_ref[...] - rm_ref[...] * s, 0.0)
    sb = lax.bitcast_convert_type(s.astype(jnp.bfloat16), jnp.uint16)
    ob = lax.bitcast_convert_type(o.astype(jnp.bfloat16), jnp.uint16)
    so = sb.astype(jnp.uint32) | (ob.astype(jnp.uint32) << 16)
    so_ref[...] = lax.bitcast_convert_type(so, jnp.int32)


def _make_tables(ep, rv, sv, rm, sm):
    return pl.pallas_call(
        _tables_body,
        in_specs=[
            pl.BlockSpec(memory_space=pltpu.SMEM),
            pl.BlockSpec(memory_space=pltpu.VMEM),
            pl.BlockSpec(memory_space=pltpu.VMEM),
            pl.BlockSpec(memory_space=pltpu.VMEM),
            pl.BlockSpec(memory_space=pltpu.VMEM),
        ],
        out_shape=jax.ShapeDtypeStruct((_NBUCKET, _FEAT), jnp.int32),
    )(ep, rv, sv, rm, sm)


@functools.partial(
    pl.kernel,
    out_type=jax.ShapeDtypeStruct((_BATCH, _FEAT), jnp.float32),
    mesh=plsc.VectorSubcoreMesh(core_axis_name="c", subcore_axis_name="s"),
    scratch_types=[
        pltpu.VMEM((_RPW,), jnp.int32),
        pltpu.VMEM((_NBUF, _CH, _FEAT), jnp.float32),
        pltpu.VMEM((_NBUF, _CH, _FEAT), jnp.int32),
        pltpu.VMEM((_NBUF, _CH, _FEAT), jnp.float32),
        pltpu.SemaphoreType.DMA,
        pltpu.SemaphoreType.DMA,
        pltpu.SemaphoreType.DMA,
        pltpu.SemaphoreType.DMA,
        pltpu.SemaphoreType.DMA,
        pltpu.SemaphoreType.DMA,
        pltpu.SemaphoreType.DMA,
        pltpu.SemaphoreType.DMA,
    ],
)
def _sc_apply(feat_hbm, lab_hbm, so_hbm, out_hbm,
              idx_v, f_v, t_v, r_v,
              si0, si1, si2, si3, so0, so1, so2, so3):
    sin = (si0, si1, si2, si3)
    sout = (so0, so1, so2, so3)
    wid = lax.axis_index("s") * _NC + lax.axis_index("c")
    base = wid * _RPW
    pltpu.sync_copy(lab_hbm.at[pl.ds(base, _RPW)], idx_v)

    def issue_in(ci, b):
        idx = idx_v.at[pl.ds(ci * _CH, _CH)]
        pltpu.async_copy(so_hbm.at[idx], t_v.at[b], sin[b])
        pltpu.async_copy(feat_hbm.at[pl.ds(base + ci * _CH, _CH)],
                         f_v.at[b], sin[b])

    def wait_in(b):
        pltpu.make_async_copy(so_hbm.at[idx_v.at[pl.ds(0, _CH)]],
                              t_v.at[b], sin[b]).wait()
        pltpu.make_async_copy(feat_hbm.at[pl.ds(base, _CH)],
                              f_v.at[b], sin[b]).wait()

    def wait_out(b):
        pltpu.make_async_copy(r_v.at[b], out_hbm.at[pl.ds(base, _CH)],
                              sout[b]).wait()

    for b in range(_NBUF):
        issue_in(b, b)

    def outer(ci2, carry):
        for b in range(_NBUF):
            ci = ci2 * _NBUF + b
            wait_in(b)

            @pl.when(ci2 > 0)
            def _():
                wait_out(b)

            def row_body(r, c2):
                for j in range(_FEAT // _L):
                    sl = (b, r, pl.ds(j * _L, _L))
                    w = t_v[sl]
                    sc = lax.bitcast_convert_type(lax.shift_left(w, jnp.int32(16)), jnp.float32)
                    of = lax.bitcast_convert_type(lax.bitwise_and(w, jnp.int32(-65536)), jnp.float32)
                    r_v[sl] = f_v[sl] * sc + of
                return c2

            lax.fori_loop(0, _CH, row_body, 0)
            pltpu.async_copy(r_v.at[b], out_hbm.at[pl.ds(base + ci * _CH, _CH)],
                             sout[b])

            @pl.when(ci + _NBUF < _NCHUNK)
            def _():
                issue_in(ci + _NBUF, b)
        return carry

    lax.fori_loop(0, _NCHUNK // _NBUF, outer, 0)
    for b in range(_NBUF):
        wait_out(b)


def kernel(features, labels, epoch,
           running_mean_last_epoch, running_var_last_epoch,
           smoothed_mean_last_epoch, smoothed_var_last_epoch):
    lab = jnp.clip(labels.reshape(-1).astype(jnp.int32), 0, _NBUCKET - 1)
    ep = jnp.asarray(epoch, jnp.int32).reshape(1, 1)
    so = _make_tables(
        ep, running_var_last_epoch, smoothed_var_last_epoch,
        running_mean_last_epoch, smoothed_mean_last_epoch)
    return _sc_apply(features, lab, so)
